# Initial kernel scaffold; baseline (speedup 1.0000x reference)
import jax
import jax.numpy as jnp
from jax.experimental import pallas as pl


def kernel(x, conv1_w, conv1_b, bn1_g, bn1_b, conv2_w, conv2_b, bn2_g, bn2_b, fc1_w, fc1_b, fc2_w, fc2_b):
    raise NotImplementedError("write your pallas kernel here")


# speedup vs baseline: 6.3716x; 6.3716x over previous
import functools

import numpy as np
import jax
import jax.numpy as jnp
from jax import lax
from jax.experimental import pallas as pl
from jax.experimental.pallas import tpu as pltpu

_BN_EPS = 1e-5
_BATCH_TILE = 512
_VMEM_LIMIT = 48 * 1024 * 1024


@functools.lru_cache(None)
def _tap_selector(k, stride, pad, h, w):
    ho = (h + 2 * pad - k) // stride + 1
    wo = (w + 2 * pad - k) // stride + 1
    t = np.zeros((k * k, ho * wo, h * w), np.float32)
    for kh in range(k):
        for kw in range(k):
            for oy in range(ho):
                for ox in range(wo):
                    iy = oy * stride + kh - pad
                    ix = ox * stride + kw - pad
                    if 0 <= iy < h and 0 <= ix < w:
                        t[kh * k + kw, oy * wo + ox, iy * w + ix] = 1.0
    return t, ho, wo


def _round_up(a, m):
    return ((a + m - 1) // m) * m


def _conv_matrix_t(conv_w, sel, cin_hw, cout_p):
    cout, cin = conv_w.shape[0], conv_w.shape[1]
    wk = conv_w.reshape(cout, cin, -1)
    m = jnp.einsum("cdt,toi->dico", wk, jnp.asarray(sel))
    return m.reshape(cin_hw, cout_p)


def _p1_body(x_ref, m1_ref, s_ref, q_ref):
    z = jnp.dot(x_ref[...], m1_ref[...], preferred_element_type=jnp.float32)
    s_ref[...] = jnp.sum(z, axis=0).reshape(1, 1, -1)
    q_ref[...] = jnp.sum(z * z, axis=0).reshape(1, 1, -1)


def _p2_body(x_ref, m1_ref, s1_ref, t1_ref, m2_ref, y_ref, s_ref, q_ref):
    z1 = jnp.dot(x_ref[...], m1_ref[...], preferred_element_type=jnp.float32)
    a1 = jnp.maximum(z1 * s1_ref[...] + t1_ref[...], 0.0)
    z2 = jnp.dot(a1, m2_ref[...], preferred_element_type=jnp.float32)
    y_ref[...] = z2
    s_ref[...] = jnp.sum(z2, axis=0).reshape(1, 1, -1)
    q_ref[...] = jnp.sum(z2 * z2, axis=0).reshape(1, 1, -1)


def _p3_body(y_ref, s2_ref, t2_ref, w1_ref, b1_ref, w2_ref, b2_ref,
             lg_ref, lp_ref):
    a2 = jnp.maximum(y_ref[...] * s2_ref[...] + t2_ref[...], 0.0)
    h = jnp.dot(a2, w1_ref[...], preferred_element_type=jnp.float32) + b1_ref[...]
    lg = jnp.dot(h, w2_ref[...], preferred_element_type=jnp.float32) + b2_ref[...]
    lg_ref[...] = lg
    m = jnp.max(lg, axis=1, keepdims=True)
    sh = lg - m
    lp_ref[...] = sh - jnp.log(jnp.sum(jnp.exp(sh), axis=1, keepdims=True))


def _bn_rows(s_part, q_part, count, gamma, beta, channels, positions, corr=None):
    s = jnp.sum(s_part, axis=(0, 1))
    q = jnp.sum(q_part, axis=(0, 1))
    if corr is not None:
        s = s - corr[0]
        q = q - corr[1]
    sc = s.reshape(channels, positions).sum(axis=1)
    qc = q.reshape(channels, positions).sum(axis=1)
    mean = sc / count
    var = qc / count - mean * mean
    scale = gamma * lax.rsqrt(var + _BN_EPS)
    shift = beta - mean * scale
    return (jnp.repeat(scale, positions).reshape(1, -1),
            jnp.repeat(shift, positions).reshape(1, -1))


def kernel(x, conv1_w, conv1_b, bn1_g, bn1_b, conv2_w, conv2_b, bn2_g, bn2_b,
           fc1_w, fc1_b, fc2_w, fc2_b):
    n, cin, h, w = x.shape
    sel1, h1, w1 = _tap_selector(3, 2, 1, h, w)
    sel2, h2, w2 = _tap_selector(3, 2, 1, h1, w1)
    c1 = conv1_w.shape[0]
    c2 = conv2_w.shape[0]
    p1 = h1 * w1
    p2 = h2 * w2
    f0 = cin * h * w
    f1 = c1 * p1
    f2 = c2 * p2

    m1t = _conv_matrix_t(conv1_w, sel1, f0, f1)
    m2t = _conv_matrix_t(conv2_w, sel2, f1, f2)

    x2d = x.reshape(n, f0).astype(jnp.float32)
    nt = _BATCH_TILE if n >= _BATCH_TILE else _round_up(n, 8)
    npad = _round_up(n, nt)
    if npad != n:
        x2d = jnp.pad(x2d, ((0, npad - n), (0, 0)))
    g = npad // nt

    params = pltpu.CompilerParams(
        dimension_semantics=("parallel",), vmem_limit_bytes=_VMEM_LIMIT)

    s1p, q1p = pl.pallas_call(
        _p1_body,
        out_shape=(jax.ShapeDtypeStruct((g, 1, f1), jnp.float32),
                   jax.ShapeDtypeStruct((g, 1, f1), jnp.float32)),
        grid=(g,),
        in_specs=[pl.BlockSpec((nt, f0), lambda i: (i, 0)),
                  pl.BlockSpec((f0, f1), lambda i: (0, 0))],
        out_specs=(pl.BlockSpec((1, 1, f1), lambda i: (i, 0, 0)),
                   pl.BlockSpec((1, 1, f1), lambda i: (i, 0, 0))),
        compiler_params=params,
    )(x2d, m1t)
    s1row, t1row = _bn_rows(s1p, q1p, float(n * p1), bn1_g, bn1_b, c1, p1)

    y2, s2p, q2p = pl.pallas_call(
        _p2_body,
        out_shape=(jax.ShapeDtypeStruct((npad, f2), jnp.float32),
                   jax.ShapeDtypeStruct((g, 1, f2), jnp.float32),
                   jax.ShapeDtypeStruct((g, 1, f2), jnp.float32)),
        grid=(g,),
        in_specs=[pl.BlockSpec((nt, f0), lambda i: (i, 0)),
                  pl.BlockSpec((f0, f1), lambda i: (0, 0)),
                  pl.BlockSpec((1, f1), lambda i: (0, 0)),
                  pl.BlockSpec((1, f1), lambda i: (0, 0)),
                  pl.BlockSpec((f1, f2), lambda i: (0, 0))],
        out_specs=(pl.BlockSpec((nt, f2), lambda i: (i, 0)),
                   pl.BlockSpec((1, 1, f2), lambda i: (i, 0, 0)),
                   pl.BlockSpec((1, 1, f2), lambda i: (i, 0, 0))),
        compiler_params=params,
    )(x2d, m1t, s1row, t1row, m2t)

    corr = None
    if npad != n:
        d = jnp.dot(jnp.maximum(t1row, 0.0), m2t)[0]
        extra = float(npad - n)
        corr = (extra * d, extra * d * d)
    s2row, t2row = _bn_rows(s2p, q2p, float(n * p2), bn2_g, bn2_b, c2, p2, corr)

    logits_p, logp_p = pl.pallas_call(
        _p3_body,
        out_shape=(jax.ShapeDtypeStruct((npad, fc2_w.shape[0]), jnp.float32),
                   jax.ShapeDtypeStruct((npad, fc2_w.shape[0]), jnp.float32)),
        grid=(g,),
        in_specs=[pl.BlockSpec((nt, f2), lambda i: (i, 0)),
                  pl.BlockSpec((1, f2), lambda i: (0, 0)),
                  pl.BlockSpec((1, f2), lambda i: (0, 0)),
                  pl.BlockSpec((f2, fc1_w.shape[0]), lambda i: (0, 0)),
                  pl.BlockSpec((1, fc1_w.shape[0]), lambda i: (0, 0)),
                  pl.BlockSpec((fc1_w.shape[0], fc2_w.shape[0]), lambda i: (0, 0)),
                  pl.BlockSpec((1, fc2_w.shape[0]), lambda i: (0, 0))],
        out_specs=(pl.BlockSpec((nt, fc2_w.shape[0]), lambda i: (i, 0)),
                   pl.BlockSpec((nt, fc2_w.shape[0]), lambda i: (i, 0))),
        compiler_params=params,
    )(y2, s2row, t2row, fc1_w.T, fc1_b.reshape(1, -1),
      fc2_w.T, fc2_b.reshape(1, -1))

    logits = logits_p[:n] if npad != n else logits_p
    logp = logp_p[:n] if npad != n else logp_p
    return {"output": logp, "logit": logits}


# speedup vs baseline: 6.9105x; 1.0846x over previous
import functools

import numpy as np
import jax
import jax.numpy as jnp
from jax import lax
from jax.experimental import pallas as pl
from jax.experimental.pallas import tpu as pltpu

_BN_EPS = 1e-5
_BATCH_TILE = 512
_VMEM_LIMIT = 48 * 1024 * 1024


@functools.lru_cache(None)
def _tap_selector(k, stride, pad, h, w):
    ho = (h + 2 * pad - k) // stride + 1
    wo = (w + 2 * pad - k) // stride + 1
    t = np.zeros((k * k, ho * wo, h * w), np.float32)
    for kh in range(k):
        for kw in range(k):
            for oy in range(ho):
                for ox in range(wo):
                    iy = oy * stride + kh - pad
                    ix = ox * stride + kw - pad
                    if 0 <= iy < h and 0 <= ix < w:
                        t[kh * k + kw, oy * wo + ox, iy * w + ix] = 1.0
    return t, ho, wo


@functools.lru_cache(None)
def _sel1_flat(k, stride, pad, h, w):
    t, ho, wo = _tap_selector(k, stride, pad, h, w)
    return t.reshape(k * k, -1), ho, wo


@functools.lru_cache(None)
def _sel2_flat(k, stride, pad, h, w, cin):
    t, ho, wo = _tap_selector(k, stride, pad, h, w)
    p, hw = ho * wo, h * w
    s = np.zeros((cin, k * k, p, cin, hw), np.float32)
    for d in range(cin):
        s[d, :, :, d, :] = t
    return s.reshape(cin * k * k, p * cin * hw), ho, wo


def _round_up(a, m):
    return ((a + m - 1) // m) * m


def _dot_t(a, b_t):
    return lax.dot_general(a, b_t, (((1,), (1,)), ((), ())),
                           preferred_element_type=jnp.float32)


def _p1_body(x_ref, m1_ref, s_ref, q_ref):
    z = _dot_t(x_ref[...], m1_ref[...])
    s_ref[...] = jnp.sum(z, axis=0).reshape(1, 1, -1)
    q_ref[...] = jnp.sum(z * z, axis=0).reshape(1, 1, -1)


def _p2_body(x_ref, m1_ref, s1_ref, t1_ref, m2_ref, y_ref, s_ref, q_ref):
    z1 = _dot_t(x_ref[...], m1_ref[...])
    a1 = jnp.maximum(z1 * s1_ref[...] + t1_ref[...], 0.0)
    z2 = _dot_t(a1, m2_ref[...])
    y_ref[...] = z2
    s_ref[...] = jnp.sum(z2, axis=0).reshape(1, 1, -1)
    q_ref[...] = jnp.sum(z2 * z2, axis=0).reshape(1, 1, -1)


def _p3_body(y_ref, s2_ref, t2_ref, w1_ref, b1_ref, w2_ref, b2_ref,
             lg_ref, lp_ref):
    a2 = jnp.maximum(y_ref[...] * s2_ref[...] + t2_ref[...], 0.0)
    h = _dot_t(a2, w1_ref[...]) + b1_ref[...]
    lg = _dot_t(h, w2_ref[...]) + b2_ref[...]
    lg_ref[...] = lg
    m = jnp.max(lg, axis=1, keepdims=True)
    sh = lg - m
    lp_ref[...] = sh - jnp.log(jnp.sum(jnp.exp(sh), axis=1, keepdims=True))


def _bn_rows(s_part, q_part, count, gamma, beta, channels, positions, corr=None):
    s = jnp.sum(s_part, axis=(0, 1))
    q = jnp.sum(q_part, axis=(0, 1))
    if corr is not None:
        s = s - corr[0]
        q = q - corr[1]
    sc = s.reshape(channels, positions).sum(axis=1)
    qc = q.reshape(channels, positions).sum(axis=1)
    mean = sc / count
    var = qc / count - mean * mean
    scale = gamma * lax.rsqrt(var + _BN_EPS)
    shift = beta - mean * scale
    return (jnp.repeat(scale, positions).reshape(1, -1),
            jnp.repeat(shift, positions).reshape(1, -1))


def kernel(x, conv1_w, conv1_b, bn1_g, bn1_b, conv2_w, conv2_b, bn2_g, bn2_b,
           fc1_w, fc1_b, fc2_w, fc2_b):
    n, cin, h, w = x.shape
    sel1, h1, w1 = _sel1_flat(3, 2, 1, h, w)
    c1 = conv1_w.shape[0]
    c2 = conv2_w.shape[0]
    sel2, h2, w2 = _sel2_flat(3, 2, 1, h1, w1, c1)
    p1 = h1 * w1
    p2 = h2 * w2
    f0 = cin * h * w
    f1 = c1 * p1
    f2 = c2 * p2

    m1 = jnp.dot(conv1_w.reshape(c1, cin * 9), jnp.asarray(sel1)).reshape(f1, f0)
    m2 = jnp.dot(conv2_w.reshape(c2, c1 * 9), jnp.asarray(sel2)).reshape(f2, f1)

    x2d = x.reshape(n, f0).astype(jnp.float32)
    nt = _BATCH_TILE if n >= _BATCH_TILE else _round_up(n, 8)
    npad = _round_up(n, nt)
    if npad != n:
        x2d = jnp.pad(x2d, ((0, npad - n), (0, 0)))
    g = npad // nt

    params = pltpu.CompilerParams(
        dimension_semantics=("parallel",), vmem_limit_bytes=_VMEM_LIMIT)

    s1p, q1p = pl.pallas_call(
        _p1_body,
        out_shape=(jax.ShapeDtypeStruct((g, 1, f1), jnp.float32),
                   jax.ShapeDtypeStruct((g, 1, f1), jnp.float32)),
        grid=(g,),
        in_specs=[pl.BlockSpec((nt, f0), lambda i: (i, 0)),
                  pl.BlockSpec((f1, f0), lambda i: (0, 0))],
        out_specs=(pl.BlockSpec((1, 1, f1), lambda i: (i, 0, 0)),
                   pl.BlockSpec((1, 1, f1), lambda i: (i, 0, 0))),
        compiler_params=params,
    )(x2d, m1)
    s1row, t1row = _bn_rows(s1p, q1p, float(n * p1), bn1_g, bn1_b, c1, p1)

    y2, s2p, q2p = pl.pallas_call(
        _p2_body,
        out_shape=(jax.ShapeDtypeStruct((npad, f2), jnp.float32),
                   jax.ShapeDtypeStruct((g, 1, f2), jnp.float32),
                   jax.ShapeDtypeStruct((g, 1, f2), jnp.float32)),
        grid=(g,),
        in_specs=[pl.BlockSpec((nt, f0), lambda i: (i, 0)),
                  pl.BlockSpec((f1, f0), lambda i: (0, 0)),
                  pl.BlockSpec((1, f1), lambda i: (0, 0)),
                  pl.BlockSpec((1, f1), lambda i: (0, 0)),
                  pl.BlockSpec((f2, f1), lambda i: (0, 0))],
        out_specs=(pl.BlockSpec((nt, f2), lambda i: (i, 0)),
                   pl.BlockSpec((1, 1, f2), lambda i: (i, 0, 0)),
                   pl.BlockSpec((1, 1, f2), lambda i: (i, 0, 0))),
        compiler_params=params,
    )(x2d, m1, s1row, t1row, m2)

    corr = None
    if npad != n:
        d = _dot_t(jnp.maximum(t1row, 0.0), m2)[0]
        extra = float(npad - n)
        corr = (extra * d, extra * d * d)
    s2row, t2row = _bn_rows(s2p, q2p, float(n * p2), bn2_g, bn2_b, c2, p2, corr)

    logits_p, logp_p = pl.pallas_call(
        _p3_body,
        out_shape=(jax.ShapeDtypeStruct((npad, fc2_w.shape[0]), jnp.float32),
                   jax.ShapeDtypeStruct((npad, fc2_w.shape[0]), jnp.float32)),
        grid=(g,),
        in_specs=[pl.BlockSpec((nt, f2), lambda i: (i, 0)),
                  pl.BlockSpec((1, f2), lambda i: (0, 0)),
                  pl.BlockSpec((1, f2), lambda i: (0, 0)),
                  pl.BlockSpec((fc1_w.shape[0], f2), lambda i: (0, 0)),
                  pl.BlockSpec((1, fc1_w.shape[0]), lambda i: (0, 0)),
                  pl.BlockSpec((fc2_w.shape[0], fc1_w.shape[0]), lambda i: (0, 0)),
                  pl.BlockSpec((1, fc2_w.shape[0]), lambda i: (0, 0))],
        out_specs=(pl.BlockSpec((nt, fc2_w.shape[0]), lambda i: (i, 0)),
                   pl.BlockSpec((nt, fc2_w.shape[0]), lambda i: (i, 0))),
        compiler_params=params,
    )(y2, s2row, t2row, fc1_w, fc1_b.reshape(1, -1),
      fc2_w, fc2_b.reshape(1, -1))

    logits = logits_p[:n] if npad != n else logits_p
    logp = logp_p[:n] if npad != n else logp_p
    return {"output": logp, "logit": logits}


# speedup vs baseline: 8.1793x; 1.1836x over previous
import functools

import numpy as np
import jax
import jax.numpy as jnp
from jax import lax
from jax.experimental import pallas as pl
from jax.experimental.pallas import tpu as pltpu

_BN_EPS = 1e-5
_BATCH_TILE = 512
_VMEM_LIMIT = 48 * 1024 * 1024


@functools.lru_cache(None)
def _tap_selector(k, stride, pad, h, w):
    ho = (h + 2 * pad - k) // stride + 1
    wo = (w + 2 * pad - k) // stride + 1
    t = np.zeros((k * k, ho * wo, h * w), np.float32)
    for kh in range(k):
        for kw in range(k):
            for oy in range(ho):
                for ox in range(wo):
                    iy = oy * stride + kh - pad
                    ix = ox * stride + kw - pad
                    if 0 <= iy < h and 0 <= ix < w:
                        t[kh * k + kw, oy * wo + ox, iy * w + ix] = 1.0
    return t, ho, wo


@functools.lru_cache(None)
def _sel1_flat(k, stride, pad, h, w):
    t, ho, wo = _tap_selector(k, stride, pad, h, w)
    return t.reshape(k * k, -1), ho, wo


@functools.lru_cache(None)
def _sel2_flat(k, stride, pad, h, w, cin):
    t, ho, wo = _tap_selector(k, stride, pad, h, w)
    p, hw = ho * wo, h * w
    s = np.zeros((cin, k * k, p, cin, hw), np.float32)
    for d in range(cin):
        s[d, :, :, d, :] = t
    return s.reshape(cin * k * k, p * cin * hw), ho, wo


def _round_up(a, m):
    return ((a + m - 1) // m) * m


def _p1_body(x_ref, m1_ref, s_ref, q_ref, g_acc, sx_acc):
    i = pl.program_id(0)
    xb = x_ref[...]

    @pl.when(i == 0)
    def _init():
        g_acc[...] = jnp.zeros_like(g_acc)
        sx_acc[...] = jnp.zeros_like(sx_acc)

    g_acc[...] += lax.dot_general(xb, xb, (((1,), (1,)), ((), ())),
                                  preferred_element_type=jnp.float32)
    sx_acc[...] += jnp.sum(xb, axis=1, keepdims=True).astype(jnp.float32)

    @pl.when(i == pl.num_programs(0) - 1)
    def _fin():
        m1 = m1_ref[...]
        e = jnp.dot(m1, g_acc[...].astype(m1.dtype),
                    preferred_element_type=jnp.float32)
        s_ref[...] = jnp.dot(m1, sx_acc[...].astype(m1.dtype),
                             preferred_element_type=jnp.float32)
        q_ref[...] = jnp.sum(e * m1.astype(jnp.float32), axis=1, keepdims=True)


def _p2_body(x_ref, m1_ref, s1_ref, t1_ref, m2_ref, y_ref, s_ref, q_ref):
    z1 = jnp.dot(m1_ref[...], x_ref[...], preferred_element_type=jnp.float32)
    a1 = jnp.maximum(z1 * s1_ref[...] + t1_ref[...], 0.0)
    z2 = jnp.dot(m2_ref[...], a1.astype(m2_ref.dtype),
                 preferred_element_type=jnp.float32)
    y_ref[...] = z2.astype(y_ref.dtype)
    s_ref[...] = jnp.sum(z2, axis=1, keepdims=True)[None]
    q_ref[...] = jnp.sum(z2 * z2, axis=1, keepdims=True)[None]


def _p3_body(y_ref, s2_ref, t2_ref, w1_ref, b1_ref, w2_ref, b2_ref,
             lg_ref, lp_ref):
    a2 = jnp.maximum(y_ref[...] * s2_ref[...] + t2_ref[...], 0.0)
    h = jnp.dot(w1_ref[...], a2, preferred_element_type=jnp.float32) + b1_ref[...]
    lg = jnp.dot(w2_ref[...], h, preferred_element_type=jnp.float32) + b2_ref[...]
    lg_ref[...] = lg.T
    m = jnp.max(lg, axis=0, keepdims=True)
    sh = lg - m
    lp = sh - jnp.log(jnp.sum(jnp.exp(sh), axis=0, keepdims=True))
    lp_ref[...] = lp.T


def _bn_cols(s, q, count, gamma, beta, channels, positions, corr=None):
    if corr is not None:
        s = s - corr[0]
        q = q - corr[1]
    sc = s.reshape(channels, positions).sum(axis=1)
    qc = q.reshape(channels, positions).sum(axis=1)
    mean = sc / count
    var = qc / count - mean * mean
    scale = gamma * lax.rsqrt(var + _BN_EPS)
    shift = beta - mean * scale
    return (jnp.repeat(scale, positions).reshape(-1, 1),
            jnp.repeat(shift, positions).reshape(-1, 1))


def kernel(x, conv1_w, conv1_b, bn1_g, bn1_b, conv2_w, conv2_b, bn2_g, bn2_b,
           fc1_w, fc1_b, fc2_w, fc2_b):
    n, cin, h, w = x.shape
    sel1, h1, w1 = _sel1_flat(3, 2, 1, h, w)
    c1 = conv1_w.shape[0]
    c2 = conv2_w.shape[0]
    sel2, h2, w2 = _sel2_flat(3, 2, 1, h1, w1, c1)
    p1 = h1 * w1
    p2 = h2 * w2
    f0 = cin * h * w
    f1 = c1 * p1
    f2 = c2 * p2
    o = fc2_w.shape[0]
    hdim = fc1_w.shape[0]

    m1 = jnp.dot(conv1_w.reshape(c1, cin * 9),
                 jnp.asarray(sel1)).reshape(f1, f0).astype(jnp.bfloat16)
    m2 = jnp.dot(conv2_w.reshape(c2, c1 * 9),
                 jnp.asarray(sel2)).reshape(f2, f1).astype(jnp.bfloat16)

    xv = jnp.transpose(x, (2, 3, 1, 0)).reshape(f0, n).astype(jnp.bfloat16)
    nt = _BATCH_TILE if n >= _BATCH_TILE else _round_up(n, 128)
    npad = _round_up(n, nt)
    if npad != n:
        xv = jnp.pad(xv, ((0, 0), (0, npad - n)))
    g = npad // nt

    params = pltpu.CompilerParams(
        dimension_semantics=("parallel",), vmem_limit_bytes=_VMEM_LIMIT)
    params_seq = pltpu.CompilerParams(
        dimension_semantics=("arbitrary",), vmem_limit_bytes=_VMEM_LIMIT)

    s1f, q1f = pl.pallas_call(
        _p1_body,
        out_shape=(jax.ShapeDtypeStruct((f1, 1), jnp.float32),
                   jax.ShapeDtypeStruct((f1, 1), jnp.float32)),
        grid=(g,),
        in_specs=[pl.BlockSpec((f0, nt), lambda i: (0, i)),
                  pl.BlockSpec((f1, f0), lambda i: (0, 0))],
        out_specs=(pl.BlockSpec((f1, 1), lambda i: (0, 0)),
                   pl.BlockSpec((f1, 1), lambda i: (0, 0))),
        scratch_shapes=[pltpu.VMEM((f0, f0), jnp.float32),
                        pltpu.VMEM((f0, 1), jnp.float32)],
        compiler_params=params_seq,
    )(xv, m1)
    s1col, t1col = _bn_cols(s1f[:, 0], q1f[:, 0], float(n * p1),
                            bn1_g, bn1_b, c1, p1)

    y2, s2p, q2p = pl.pallas_call(
        _p2_body,
        out_shape=(jax.ShapeDtypeStruct((f2, npad), jnp.bfloat16),
                   jax.ShapeDtypeStruct((g, f2, 1), jnp.float32),
                   jax.ShapeDtypeStruct((g, f2, 1), jnp.float32)),
        grid=(g,),
        in_specs=[pl.BlockSpec((f0, nt), lambda i: (0, i)),
                  pl.BlockSpec((f1, f0), lambda i: (0, 0)),
                  pl.BlockSpec((f1, 1), lambda i: (0, 0)),
                  pl.BlockSpec((f1, 1), lambda i: (0, 0)),
                  pl.BlockSpec((f2, f1), lambda i: (0, 0))],
        out_specs=(pl.BlockSpec((f2, nt), lambda i: (0, i)),
                   pl.BlockSpec((1, f2, 1), lambda i: (i, 0, 0)),
                   pl.BlockSpec((1, f2, 1), lambda i: (i, 0, 0))),
        compiler_params=params,
    )(xv, m1, s1col, t1col, m2)

    corr = None
    if npad != n:
        d = jnp.dot(m2.astype(jnp.float32), jnp.maximum(t1col, 0.0))[:, 0]
        extra = float(npad - n)
        corr = (extra * d, extra * d * d)
    s2col, t2col = _bn_cols(jnp.sum(s2p, axis=(0, 2)), jnp.sum(q2p, axis=(0, 2)),
                            float(n * p2), bn2_g, bn2_b, c2, p2, corr)

    logits_p, logp_p = pl.pallas_call(
        _p3_body,
        out_shape=(jax.ShapeDtypeStruct((npad, o), jnp.float32),
                   jax.ShapeDtypeStruct((npad, o), jnp.float32)),
        grid=(g,),
        in_specs=[pl.BlockSpec((f2, nt), lambda i: (0, i)),
                  pl.BlockSpec((f2, 1), lambda i: (0, 0)),
                  pl.BlockSpec((f2, 1), lambda i: (0, 0)),
                  pl.BlockSpec((hdim, f2), lambda i: (0, 0)),
                  pl.BlockSpec((hdim, 1), lambda i: (0, 0)),
                  pl.BlockSpec((o, hdim), lambda i: (0, 0)),
                  pl.BlockSpec((o, 1), lambda i: (0, 0))],
        out_specs=(pl.BlockSpec((nt, o), lambda i: (i, 0)),
                   pl.BlockSpec((nt, o), lambda i: (i, 0))),
        compiler_params=params,
    )(y2, s2col, t2col, fc1_w, fc1_b.reshape(-1, 1),
      fc2_w, fc2_b.reshape(-1, 1))

    logits = logits_p[:n] if npad != n else logits_p
    logp = logp_p[:n] if npad != n else logp_p
    return {"output": logp, "logit": logits}
